# baseline (speedup 1.0000x reference)
import functools

import jax
import jax.numpy as jnp
from jax import lax
from jax.experimental import pallas as pl
from jax.experimental.pallas import tpu as pltpu

_B = 32
_N = 17
_HW = 128 * 128
_C = 128
_NCHUNK = _HW // _C


def _loss_kernel(hm_ref, off_ref, gt_ref, out_ref):
    i = pl.program_id(0)

    def body(f, carry):
        run_max, run_f, run_ox, run_oy = carry
        sl = pl.ds(f * _C, _C)
        hm_c = hm_ref[0, :, sl]
        ox_c = off_ref[0, 0, sl]
        oy_c = off_ref[0, 1, sl]
        upd = hm_c > run_max
        run_max = jnp.where(upd, hm_c, run_max)
        run_f = jnp.where(upd, f, run_f)
        run_ox = jnp.where(upd, ox_c, run_ox)
        run_oy = jnp.where(upd, oy_c, run_oy)
        return run_max, run_f, run_ox, run_oy

    init = (
        jnp.full((_N, _C), -jnp.inf, jnp.float32),
        jnp.zeros((_N, _C), jnp.int32),
        jnp.zeros((_N, _C), jnp.float32),
        jnp.zeros((_N, _C), jnp.float32),
    )
    carry = init
    for f in range(_NCHUNK):
        carry = body(f, carry)
    run_max, run_f, run_ox, run_oy = carry

    m = jnp.max(run_max, axis=-1, keepdims=True)
    lane_iota = lax.broadcasted_iota(jnp.int32, (_N, _C), 1)
    flat = run_f * _C + lane_iota
    masked_flat = jnp.where(run_max == m, flat, jnp.int32(_HW))
    win_flat = jnp.min(masked_flat, axis=-1, keepdims=True)
    win = masked_flat == win_flat
    ox = jnp.sum(jnp.where(win, run_ox, 0.0), axis=-1)
    oy = jnp.sum(jnp.where(win, run_oy, 0.0), axis=-1)

    gt = gt_ref[0]
    partial = jnp.sum(jnp.abs(ox - gt[:, 0]) + jnp.abs(oy - gt[:, 1]))

    @pl.when(i == 0)
    def _init():
        out_ref[0] = 0.0

    out_ref[0] += partial

    @pl.when(i == _B - 1)
    def _finish():
        out_ref[0] = out_ref[0] * (1.0 / (_B * _N * 2 * _N))


@functools.partial(jax.jit)
def _run(hm_flat, off_flat, offset_gt):
    out = pl.pallas_call(
        _loss_kernel,
        grid=(_B,),
        in_specs=[
            pl.BlockSpec((1, _N, _HW), lambda i: (i, 0, 0)),
            pl.BlockSpec((1, 2, _HW), lambda i: (i, 0, 0)),
            pl.BlockSpec((1, _N, 2), lambda i: (i, 0, 0)),
        ],
        out_specs=pl.BlockSpec(memory_space=pltpu.MemorySpace.SMEM),
        out_shape=jax.ShapeDtypeStruct((1,), jnp.float32),
    )(hm_flat, off_flat, offset_gt)
    return out[0]


def kernel(offset_map_pred, hm_gt, offset_gt):
    b, n = hm_gt.shape[0], hm_gt.shape[1]
    hm_flat = hm_gt.reshape(b, n, -1)
    off_flat = offset_map_pred.reshape(b, 2, -1)
    return _run(hm_flat, off_flat, offset_gt)


# speedup vs baseline: 2.3446x; 2.3446x over previous
import functools

import jax
import jax.numpy as jnp
from jax import lax
from jax.experimental import pallas as pl
from jax.experimental.pallas import tpu as pltpu

_B = 32
_N = 17
_H = 128
_W = 128
_S = 8
_NSLAB = _H // _S


def _loss_kernel(hm_ref, off_ref, gt_ref, out_ref):
    i = pl.program_id(0)

    run_max = jnp.full((_N, _S, _W), -jnp.inf, jnp.float32)
    run_rt = jnp.zeros((_N, _S, _W), jnp.int32)
    run_ox = jnp.zeros((_N, _S, _W), jnp.float32)
    run_oy = jnp.zeros((_N, _S, _W), jnp.float32)

    for rt in range(_NSLAB):
        sl = pl.ds(rt * _S, _S)
        hm_s = hm_ref[0, :, sl, :]
        ox_s = off_ref[0, 0, sl, :]
        oy_s = off_ref[0, 1, sl, :]
        upd = hm_s > run_max
        run_max = jnp.where(upd, hm_s, run_max)
        run_rt = jnp.where(upd, rt, run_rt)
        run_ox = jnp.where(upd, ox_s, run_ox)
        run_oy = jnp.where(upd, oy_s, run_oy)

    sub_iota = lax.broadcasted_iota(jnp.int32, (_N, _S, _W), 1)
    lane_iota = lax.broadcasted_iota(jnp.int32, (_N, _S, _W), 2)
    sl_const = sub_iota * _W + lane_iota
    flat = run_rt * (_S * _W) + sl_const

    m = jnp.max(run_max, axis=(1, 2), keepdims=True)
    masked_flat = jnp.where(run_max == m, flat, jnp.int32(_H * _W))
    win_flat = jnp.min(masked_flat, axis=(1, 2), keepdims=True)
    win = masked_flat == win_flat
    ox = jnp.sum(jnp.where(win, run_ox, 0.0), axis=(1, 2))
    oy = jnp.sum(jnp.where(win, run_oy, 0.0), axis=(1, 2))

    gt = gt_ref[0]
    partial = jnp.sum(jnp.abs(ox - gt[:, 0]) + jnp.abs(oy - gt[:, 1]))

    @pl.when(i == 0)
    def _init():
        out_ref[0] = 0.0

    out_ref[0] += partial

    @pl.when(i == _B - 1)
    def _finish():
        out_ref[0] = out_ref[0] * (1.0 / (_B * _N * 2 * _N))


@functools.partial(jax.jit)
def _run(hm_gt, offset_map_pred, offset_gt):
    out = pl.pallas_call(
        _loss_kernel,
        grid=(_B,),
        in_specs=[
            pl.BlockSpec((1, _N, _H, _W), lambda i: (i, 0, 0, 0)),
            pl.BlockSpec((1, 2, _H, _W), lambda i: (i, 0, 0, 0)),
            pl.BlockSpec((1, _N, 2), lambda i: (i, 0, 0)),
        ],
        out_specs=pl.BlockSpec(memory_space=pltpu.MemorySpace.SMEM),
        out_shape=jax.ShapeDtypeStruct((1,), jnp.float32),
    )(hm_gt, offset_map_pred, offset_gt)
    return out[0]


def kernel(offset_map_pred, hm_gt, offset_gt):
    return _run(hm_gt, offset_map_pred, offset_gt)
